# baseline (speedup 1.0000x reference)
import functools

import jax
import jax.numpy as jnp
from jax import lax
from jax.experimental import pallas as pl
from jax.experimental.pallas import tpu as pltpu
from jax.experimental.pallas import tpu_sc as plsc

N = 10000
E = 320000
NFEAT = 128
H = 8
C1 = 8
HC = H * C1
NCLASS = 7

NWORKERS = 32
EPW = E // NWORKERS
T = 80
NCHUNK = EPW // T
RPT = 624
ZROWS = 104

W1COLS = 80
W2COLS = 16


def _vperm(v, pat):
    return lax.gather(
        v, pat[:, None],
        lax.GatherDimensionNumbers(offset_dims=(), collapsed_slice_dims=(0,),
                                   start_index_map=(0,)),
        slice_sizes=(1,), mode=lax.GatherScatterMode.PROMISE_IN_BOUNDS)


def _prep1_body(x_ref, w1_ref, asrc_ref, adst_ref, xpe_ref, adst_out_ref):
    x = x_ref[...]
    w1 = w1_ref[...]
    xp = jnp.dot(x, w1, preferred_element_type=jnp.float32)
    rowh = lax.broadcasted_iota(jnp.int32, (HC, H), 0) // C1
    colh = lax.broadcasted_iota(jnp.int32, (HC, H), 1)
    mask = (rowh == colh).astype(jnp.float32)
    As = jnp.tile(asrc_ref[...].T, (H, 1)) * mask
    Ad = jnp.tile(adst_ref[...].T, (H, 1)) * mask
    asrc = jnp.dot(xp, As, preferred_element_type=jnp.float32)
    adst = jnp.dot(xp, Ad, preferred_element_type=jnp.float32)
    xpe_ref[:, 0:HC] = xp
    xpe_ref[:, HC:HC + H] = jnp.ones((N, H), jnp.float32)
    xpe_ref[:, HC + H:W1COLS] = asrc
    adst_out_ref[:, 0:H] = adst
    adst_out_ref[:, H:16] = jnp.zeros((N, 8), jnp.float32)


def _prep1(x, W1, a_src1, a_dst1):
    return pl.pallas_call(
        _prep1_body,
        out_shape=(
            jax.ShapeDtypeStruct((N, W1COLS), jnp.float32),
            jax.ShapeDtypeStruct((N, 16), jnp.float32),
        ),
    )(x, W1, a_src1, a_dst1)


def _edges1_body(src_hbm, dst_hbm, xpe_hbm, adstt_hbm, acc_out,
                 src_all, dst_all, rows0, rows1, adr0, adr1, msg0, msg1,
                 zb_v, acc_s, gsem0, gsem1, ssem0, ssem1):
    cid = lax.axis_index("c")
    sid = lax.axis_index("s")
    wid = sid * 2 + cid

    iota = lax.iota(jnp.int32, 16)
    half = iota // 8
    oct8 = iota % 8

    rows = (rows0, rows1)
    adr = (adr0, adr1)
    msg = (msg0, msg1)
    gsem = (gsem0, gsem1)
    ssem = (ssem0, ssem1)

    pltpu.sync_copy(src_hbm.at[pl.ds(wid * NCHUNK, NCHUNK)], src_all)
    pltpu.sync_copy(dst_hbm.at[pl.ds(wid * NCHUNK, NCHUNK)], dst_all)

    def _zb_row(r, _):
        for k in range(W1COLS // 16):
            zb_v[r, pl.ds(k * 16, 16)] = jnp.zeros((16,), jnp.float32)
        return 0
    lax.fori_loop(0, ZROWS, _zb_row, 0)
    for j in range(RPT // ZROWS):
        pltpu.sync_copy(zb_v, acc_s.at[pl.ds(sid * RPT + j * ZROWS, ZROWS)])

    @pl.when(sid == 0)
    def _():
        pltpu.sync_copy(zb_v.at[pl.ds(0, 16)], acc_s.at[pl.ds(16 * RPT, 16)])
    plsc.subcore_barrier()

    def start_gathers(c, b):
        pltpu.async_copy(xpe_hbm.at[src_all.at[c]], rows[b], gsem[b])
        pltpu.async_copy(adstt_hbm.at[dst_all.at[c]], adr[b], gsem[b])

    def chunk_step(c, b, wait_scatter, prefetch_c):
        pltpu.make_async_copy(xpe_hbm.at[src_all.at[c]], rows[b], gsem[b]).wait()
        pltpu.make_async_copy(adstt_hbm.at[dst_all.at[c]], adr[b], gsem[b]).wait()
        if wait_scatter:
            pltpu.make_async_copy(msg[b], acc_s.at[dst_all.at[c]], ssem[b]).wait()

        rows_b, adr_b, msg_b = rows[b], adr[b], msg[b]
        asrc_col = jnp.full((16,), HC + H, jnp.int32) + oct8

        @plsc.parallel_loop(0, T // 2, unroll=4)
        def _pair_edges(i):
            e = i * 2
            idx_r = jnp.full((16,), e, jnp.int32) + half
            va = plsc.load_gather(rows_b, [idx_r, asrc_col])
            vb = plsc.load_gather(adr_b, [idx_r, oct8])
            s = va + vb
            s = jnp.where(s > 0, s, 0.2 * s)
            w2 = jnp.exp(s)
            for k in range(5):
                pat = oct8 if k == 4 else 2 * k + half
                msg_b[e, pl.ds(k * 16, 16)] = (
                    rows_b[e, pl.ds(k * 16, 16)] * _vperm(w2, pat))
                msg_b[e + 1, pl.ds(k * 16, 16)] = (
                    rows_b[e + 1, pl.ds(k * 16, 16)] * _vperm(w2, 8 + pat))

        pltpu.async_copy(msg_b, acc_s.at[dst_all.at[c]], ssem[b], add=True)
        if prefetch_c is not None:
            start_gathers(prefetch_c, b)

    start_gathers(0, 0)
    start_gathers(1, 1)
    chunk_step(0, 0, False, 2)
    chunk_step(1, 1, False, 3)

    def _pair(g, _):
        chunk_step(2 * g, 0, True, 2 * g + 2)
        chunk_step(2 * g + 1, 1, True, 2 * g + 3)
        return 0
    lax.fori_loop(1, 61, _pair, 0)
    chunk_step(122, 0, True, 124)
    chunk_step(123, 1, True, None)
    chunk_step(124, 0, True, None)
    pltpu.make_async_copy(msg[1], acc_s.at[dst_all.at[123]], ssem[1]).wait()
    pltpu.make_async_copy(msg[0], acc_s.at[dst_all.at[124]], ssem[0]).wait()

    plsc.subcore_barrier()
    pltpu.sync_copy(acc_s.at[pl.ds(sid * RPT, RPT)],
                    acc_out.at[cid, pl.ds(sid * RPT, RPT)])

    @pl.when(sid == 0)
    def _():
        pltpu.sync_copy(acc_s.at[pl.ds(16 * RPT, 16)],
                        acc_out.at[cid, pl.ds(16 * RPT, 16)])


def _edges1(src2d, dst2d, xpe1, adst_t):
    mesh = plsc.VectorSubcoreMesh(core_axis_name="c", subcore_axis_name="s")
    f = pl.kernel(
        _edges1_body,
        out_type=jax.ShapeDtypeStruct((2, N, W1COLS), jnp.float32),
        mesh=mesh,
        compiler_params=pltpu.CompilerParams(
            use_tc_tiling_on_sc=False, needs_layout_passes=False),
        scratch_types=[
            pltpu.VMEM((NCHUNK, T), jnp.int32),
            pltpu.VMEM((NCHUNK, T), jnp.int32),
            pltpu.VMEM((T, W1COLS), jnp.float32),
            pltpu.VMEM((T, W1COLS), jnp.float32),
            pltpu.VMEM((T, 16), jnp.float32),
            pltpu.VMEM((T, 16), jnp.float32),
            pltpu.VMEM((T, W1COLS), jnp.float32),
            pltpu.VMEM((T, W1COLS), jnp.float32),
            pltpu.VMEM((ZROWS, W1COLS), jnp.float32),
            pltpu.VMEM_SHARED((N, W1COLS), jnp.float32),
            pltpu.SemaphoreType.DMA,
            pltpu.SemaphoreType.DMA,
            pltpu.SemaphoreType.DMA,
            pltpu.SemaphoreType.DMA,
        ],
    )
    return f(src2d, dst2d, xpe1, adst_t)


def _combine1_body(acc_ref, xpe_ref, adstt_ref, b1_ref, w2_ref,
                   asrc2_ref, adst2_ref, xpe2_ref, a2s_ref, a2d_ref):
    xp = xpe_ref[:, 0:HC]
    asrc = xpe_ref[:, HC + H:W1COLS]
    adst = adstt_ref[:, 0:H]
    s = asrc + adst
    wself = jnp.exp(jnp.where(s > 0, s, 0.2 * s))
    rowh = lax.broadcasted_iota(jnp.int32, (H, HC), 0)
    colh = lax.broadcasted_iota(jnp.int32, (H, HC), 1) // C1
    R = (rowh == colh).astype(jnp.float32)
    wrep = jnp.dot(wself, R, preferred_element_type=jnp.float32)
    num = acc_ref[0, :, 0:HC] + acc_ref[1, :, 0:HC] + xp * wrep
    den = acc_ref[0, :, HC:HC + H] + acc_ref[1, :, HC:HC + H] + wself
    denr = jnp.dot(den, R, preferred_element_type=jnp.float32)
    h = num / (denr + 1e-16) + b1_ref[...][None, :]

    x2p = jnp.dot(h, w2_ref[...], preferred_element_type=jnp.float32)
    a2s = jnp.dot(x2p, asrc2_ref[...].T, preferred_element_type=jnp.float32)
    a2d = jnp.dot(x2p, adst2_ref[...].T, preferred_element_type=jnp.float32)
    xpe2_ref[:, 0:NCLASS] = x2p
    xpe2_ref[:, NCLASS:8] = jnp.zeros((N, 1), jnp.float32)
    xpe2_ref[:, 8:9] = jnp.ones((N, 1), jnp.float32)
    xpe2_ref[:, 9:16] = jnp.zeros((N, 7), jnp.float32)
    a2s_ref[...] = a2s
    a2d_ref[...] = a2d


def _combine1(acc1, xpe1, adst_t, b1, W2, a_src2, a_dst2):
    return pl.pallas_call(
        _combine1_body,
        out_shape=(
            jax.ShapeDtypeStruct((N, W2COLS), jnp.float32),
            jax.ShapeDtypeStruct((N, 1), jnp.float32),
            jax.ShapeDtypeStruct((N, 1), jnp.float32),
        ),
    )(acc1, xpe1, adst_t, b1, W2, a_src2, a_dst2)


def _edges2_body(src_hbm, dst_hbm, xpe2_hbm, a2s_hbm, a2d_hbm, acc_out,
                 src_all, dst_all, rows0, rows1, msg0, msg1,
                 a2s_v, a2d_v, zb_v, acc_s, gsem0, gsem1, ssem0, ssem1):
    cid = lax.axis_index("c")
    sid = lax.axis_index("s")
    wid = sid * 2 + cid

    rows = (rows0, rows1)
    msg = (msg0, msg1)
    gsem = (gsem0, gsem1)
    ssem = (ssem0, ssem1)

    pltpu.sync_copy(a2s_hbm, a2s_v)
    pltpu.sync_copy(a2d_hbm, a2d_v)
    pltpu.sync_copy(src_hbm.at[pl.ds(wid * NCHUNK, NCHUNK)], src_all)
    pltpu.sync_copy(dst_hbm.at[pl.ds(wid * NCHUNK, NCHUNK)], dst_all)

    def _zb_row(r, _):
        zb_v[r, pl.ds(0, 16)] = jnp.zeros((16,), jnp.float32)
        return 0
    lax.fori_loop(0, ZROWS, _zb_row, 0)
    for j in range(RPT // ZROWS):
        pltpu.sync_copy(zb_v, acc_s.at[pl.ds(sid * RPT + j * ZROWS, ZROWS)])

    @pl.when(sid == 0)
    def _():
        pltpu.sync_copy(zb_v.at[pl.ds(0, 16)], acc_s.at[pl.ds(16 * RPT, 16)])
    plsc.subcore_barrier()

    def start_gather(c, b):
        pltpu.async_copy(xpe2_hbm.at[src_all.at[c]], rows[b], gsem[b])

    def chunk_step(c, b, wait_scatter, prefetch_c):
        pltpu.make_async_copy(xpe2_hbm.at[src_all.at[c]], rows[b], gsem[b]).wait()
        if wait_scatter:
            pltpu.make_async_copy(msg[b], acc_s.at[dst_all.at[c]], ssem[b]).wait()
        rows_b, msg_b = rows[b], msg[b]

        @plsc.parallel_loop(0, T // 16, unroll=1)
        def _grp(j):
            sidx = src_all[c, pl.ds(j * 16, 16)]
            didx = dst_all[c, pl.ds(j * 16, 16)]
            s = plsc.load_gather(a2s_v, [sidx]) + plsc.load_gather(a2d_v, [didx])
            s = jnp.where(s > 0, s, 0.2 * s)
            w16 = jnp.exp(s)
            e0 = j * 16
            for l in range(16):
                msg_b[e0 + l, pl.ds(0, 16)] = (
                    rows_b[e0 + l, pl.ds(0, 16)]
                    * _vperm(w16, jnp.full((16,), l, jnp.int32)))

        pltpu.async_copy(msg_b, acc_s.at[dst_all.at[c]], ssem[b], add=True)
        if prefetch_c is not None:
            start_gather(prefetch_c, b)

    start_gather(0, 0)
    start_gather(1, 1)
    chunk_step(0, 0, False, 2)
    chunk_step(1, 1, False, 3)

    def _pair(g, _):
        chunk_step(2 * g, 0, True, 2 * g + 2)
        chunk_step(2 * g + 1, 1, True, 2 * g + 3)
        return 0
    lax.fori_loop(1, 61, _pair, 0)
    chunk_step(122, 0, True, 124)
    chunk_step(123, 1, True, None)
    chunk_step(124, 0, True, None)
    pltpu.make_async_copy(msg[1], acc_s.at[dst_all.at[123]], ssem[1]).wait()
    pltpu.make_async_copy(msg[0], acc_s.at[dst_all.at[124]], ssem[0]).wait()

    plsc.subcore_barrier()
    pltpu.sync_copy(acc_s.at[pl.ds(sid * RPT, RPT)],
                    acc_out.at[cid, pl.ds(sid * RPT, RPT)])

    @pl.when(sid == 0)
    def _():
        pltpu.sync_copy(acc_s.at[pl.ds(16 * RPT, 16)],
                        acc_out.at[cid, pl.ds(16 * RPT, 16)])


def _edges2(src2d, dst2d, xpe2, a2s, a2d):
    mesh = plsc.VectorSubcoreMesh(core_axis_name="c", subcore_axis_name="s")
    f = pl.kernel(
        _edges2_body,
        out_type=jax.ShapeDtypeStruct((2, N, W2COLS), jnp.float32),
        mesh=mesh,
        compiler_params=pltpu.CompilerParams(
            use_tc_tiling_on_sc=False, needs_layout_passes=False),
        scratch_types=[
            pltpu.VMEM((NCHUNK, T), jnp.int32),
            pltpu.VMEM((NCHUNK, T), jnp.int32),
            pltpu.VMEM((T, W2COLS), jnp.float32),
            pltpu.VMEM((T, W2COLS), jnp.float32),
            pltpu.VMEM((T, W2COLS), jnp.float32),
            pltpu.VMEM((T, W2COLS), jnp.float32),
            pltpu.VMEM((N,), jnp.float32),
            pltpu.VMEM((N,), jnp.float32),
            pltpu.VMEM((ZROWS, W2COLS), jnp.float32),
            pltpu.VMEM_SHARED((N, W2COLS), jnp.float32),
            pltpu.SemaphoreType.DMA,
            pltpu.SemaphoreType.DMA,
            pltpu.SemaphoreType.DMA,
            pltpu.SemaphoreType.DMA,
        ],
    )
    return f(src2d, dst2d, xpe2, a2s, a2d)


def _final_body(acc_ref, xpe2_ref, a2s_ref, a2d_ref, b2_ref, out_ref):
    x2p = xpe2_ref[:, 0:NCLASS]
    s = a2s_ref[...] + a2d_ref[...]
    wself = jnp.exp(jnp.where(s > 0, s, 0.2 * s))
    num = acc_ref[0, :, 0:NCLASS] + acc_ref[1, :, 0:NCLASS] + x2p * wself
    den = acc_ref[0, :, 8:9] + acc_ref[1, :, 8:9] + wself
    o = num / (den + 1e-16) + b2_ref[...][None, :]
    o = jnp.where(o > 0, o, jnp.exp(o) - 1.0)
    m = jnp.max(o, axis=1, keepdims=True)
    t = o - m
    lse = jnp.log(jnp.sum(jnp.exp(t), axis=1, keepdims=True))
    out_ref[...] = t - lse


def _final(acc2, xpe2, a2s, a2d, b2):
    return pl.pallas_call(
        _final_body,
        out_shape=jax.ShapeDtypeStruct((N, NCLASS), jnp.float32),
    )(acc2, xpe2, a2s, a2d, b2)


def kernel(x, edge_index, W1, a_src1, a_dst1, b1, W2, a_src2, a_dst2, b2):
    src2d = edge_index[0].reshape(NWORKERS * NCHUNK, T)
    dst2d = edge_index[1].reshape(NWORKERS * NCHUNK, T)
    xpe1, adst_t = _prep1(x, W1, a_src1, a_dst1)
    acc1 = _edges1(src2d, dst2d, xpe1, adst_t)
    xpe2, a2s, a2d = _combine1(acc1, xpe1, adst_t, b1, W2, a_src2, a_dst2)
    acc2 = _edges2(src2d, dst2d, xpe2, a2s.reshape(N), a2d.reshape(N))
    return _final(acc2, xpe2, a2s, a2d, b2)


# speedup vs baseline: 1.1235x; 1.1235x over previous
import functools

import jax
import jax.numpy as jnp
from jax import lax
from jax.experimental import pallas as pl
from jax.experimental.pallas import tpu as pltpu
from jax.experimental.pallas import tpu_sc as plsc

N = 10000
E = 320000
NFEAT = 128
H = 8
C1 = 8
HC = H * C1
NCLASS = 7

NWORKERS = 32
EPW = E // NWORKERS
T = 80
NCHUNK = EPW // T
RPT = 624
ZROWS = 104

W1COLS = 80
W2COLS = 16


def _vperm(v, pat):
    return lax.gather(
        v, pat[:, None],
        lax.GatherDimensionNumbers(offset_dims=(), collapsed_slice_dims=(0,),
                                   start_index_map=(0,)),
        slice_sizes=(1,), mode=lax.GatherScatterMode.PROMISE_IN_BOUNDS)


def _prep1_body(x_ref, w1_ref, asrc_ref, adst_ref, xpe_ref, adst_out_ref):
    x = x_ref[...]
    w1 = w1_ref[...]
    xp = jnp.dot(x, w1, preferred_element_type=jnp.float32)
    rowh = lax.broadcasted_iota(jnp.int32, (HC, H), 0) // C1
    colh = lax.broadcasted_iota(jnp.int32, (HC, H), 1)
    mask = (rowh == colh).astype(jnp.float32)
    As = jnp.tile(asrc_ref[...].T, (H, 1)) * mask
    Ad = jnp.tile(adst_ref[...].T, (H, 1)) * mask
    asrc = jnp.dot(xp, As, preferred_element_type=jnp.float32)
    adst = jnp.dot(xp, Ad, preferred_element_type=jnp.float32)
    xpe_ref[:, 0:HC] = xp
    xpe_ref[:, HC:HC + H] = jnp.ones((N, H), jnp.float32)
    xpe_ref[:, HC + H:W1COLS] = asrc
    adst_out_ref[:, 0:H] = adst
    adst_out_ref[:, H:16] = jnp.zeros((N, 8), jnp.float32)


def _prep1(x, W1, a_src1, a_dst1):
    return pl.pallas_call(
        _prep1_body,
        out_shape=(
            jax.ShapeDtypeStruct((N, W1COLS), jnp.float32),
            jax.ShapeDtypeStruct((N, 16), jnp.float32),
        ),
    )(x, W1, a_src1, a_dst1)


def _edges1_body(src_hbm, dst_hbm, xpe_hbm, adstt_hbm, acc_out,
                 src_all, dst_all, rows0, rows1, rows2, adr0, adr1, adr2,
                 msg0, msg1, msg2, zb_v, acc_s,
                 gsem0, gsem1, gsem2, ssem0, ssem1, ssem2):
    cid = lax.axis_index("c")
    sid = lax.axis_index("s")
    wid = sid * 2 + cid

    iota = lax.iota(jnp.int32, 16)
    half = iota // 8
    oct8 = iota % 8

    rows = (rows0, rows1, rows2)
    adr = (adr0, adr1, adr2)
    msg = (msg0, msg1, msg2)
    gsem = (gsem0, gsem1, gsem2)
    ssem = (ssem0, ssem1, ssem2)

    pltpu.sync_copy(src_hbm.at[pl.ds(wid * NCHUNK, NCHUNK)], src_all)
    pltpu.sync_copy(dst_hbm.at[pl.ds(wid * NCHUNK, NCHUNK)], dst_all)

    def _zb_row(r, _):
        for k in range(W1COLS // 16):
            zb_v[r, pl.ds(k * 16, 16)] = jnp.zeros((16,), jnp.float32)
        return 0
    lax.fori_loop(0, ZROWS, _zb_row, 0)
    for j in range(RPT // ZROWS):
        pltpu.sync_copy(zb_v, acc_s.at[pl.ds(sid * RPT + j * ZROWS, ZROWS)])

    @pl.when(sid == 0)
    def _():
        pltpu.sync_copy(zb_v.at[pl.ds(0, 16)], acc_s.at[pl.ds(16 * RPT, 16)])
    plsc.subcore_barrier()

    def start_gathers(c, b):
        pltpu.async_copy(xpe_hbm.at[src_all.at[c]], rows[b], gsem[b])
        pltpu.async_copy(adstt_hbm.at[dst_all.at[c]], adr[b], gsem[b])

    def chunk_step(c, b, wait_scatter, prefetch_c):
        pltpu.make_async_copy(xpe_hbm.at[src_all.at[c]], rows[b], gsem[b]).wait()
        pltpu.make_async_copy(adstt_hbm.at[dst_all.at[c]], adr[b], gsem[b]).wait()
        if wait_scatter:
            pltpu.make_async_copy(msg[b], acc_s.at[dst_all.at[c]], ssem[b]).wait()

        rows_b, adr_b, msg_b = rows[b], adr[b], msg[b]
        asrc_col = jnp.full((16,), HC + H, jnp.int32) + oct8

        @plsc.parallel_loop(0, T // 2, unroll=4)
        def _pair_edges(i):
            e = i * 2
            idx_r = jnp.full((16,), e, jnp.int32) + half
            va = plsc.load_gather(rows_b, [idx_r, asrc_col])
            vb = plsc.load_gather(adr_b, [idx_r, oct8])
            s = va + vb
            s = jnp.where(s > 0, s, 0.2 * s)
            w2 = jnp.exp(s)
            for k in range(5):
                pat = oct8 if k == 4 else 2 * k + half
                msg_b[e, pl.ds(k * 16, 16)] = (
                    rows_b[e, pl.ds(k * 16, 16)] * _vperm(w2, pat))
                msg_b[e + 1, pl.ds(k * 16, 16)] = (
                    rows_b[e + 1, pl.ds(k * 16, 16)] * _vperm(w2, 8 + pat))

        pltpu.async_copy(msg_b, acc_s.at[dst_all.at[c]], ssem[b], add=True)
        if prefetch_c is not None:
            start_gathers(prefetch_c, b)

    start_gathers(0, 0)
    start_gathers(1, 1)
    start_gathers(2, 2)
    chunk_step(0, 0, False, 3)
    chunk_step(1, 1, False, 4)
    chunk_step(2, 2, False, 5)

    def _trip(g, _):
        chunk_step(3 * g, 0, True, 3 * g + 3)
        chunk_step(3 * g + 1, 1, True, 3 * g + 4)
        chunk_step(3 * g + 2, 2, True, 3 * g + 5)
        return 0
    lax.fori_loop(1, 40, _trip, 0)
    chunk_step(120, 0, True, 123)
    chunk_step(121, 1, True, 124)
    chunk_step(122, 2, True, None)
    chunk_step(123, 0, True, None)
    chunk_step(124, 1, True, None)
    pltpu.make_async_copy(msg[2], acc_s.at[dst_all.at[122]], ssem[2]).wait()
    pltpu.make_async_copy(msg[0], acc_s.at[dst_all.at[123]], ssem[0]).wait()
    pltpu.make_async_copy(msg[1], acc_s.at[dst_all.at[124]], ssem[1]).wait()

    plsc.subcore_barrier()
    pltpu.sync_copy(acc_s.at[pl.ds(sid * RPT, RPT)],
                    acc_out.at[cid, pl.ds(sid * RPT, RPT)])

    @pl.when(sid == 0)
    def _():
        pltpu.sync_copy(acc_s.at[pl.ds(16 * RPT, 16)],
                        acc_out.at[cid, pl.ds(16 * RPT, 16)])


def _edges1(src2d, dst2d, xpe1, adst_t):
    mesh = plsc.VectorSubcoreMesh(core_axis_name="c", subcore_axis_name="s")
    f = pl.kernel(
        _edges1_body,
        out_type=jax.ShapeDtypeStruct((2, N, W1COLS), jnp.float32),
        mesh=mesh,
        compiler_params=pltpu.CompilerParams(
            use_tc_tiling_on_sc=False, needs_layout_passes=False),
        scratch_types=[
            pltpu.VMEM((NCHUNK, T), jnp.int32),
            pltpu.VMEM((NCHUNK, T), jnp.int32),
            pltpu.VMEM((T, W1COLS), jnp.float32),
            pltpu.VMEM((T, W1COLS), jnp.float32),
            pltpu.VMEM((T, W1COLS), jnp.float32),
            pltpu.VMEM((T, 16), jnp.float32),
            pltpu.VMEM((T, 16), jnp.float32),
            pltpu.VMEM((T, 16), jnp.float32),
            pltpu.VMEM((T, W1COLS), jnp.float32),
            pltpu.VMEM((T, W1COLS), jnp.float32),
            pltpu.VMEM((T, W1COLS), jnp.float32),
            pltpu.VMEM((ZROWS, W1COLS), jnp.float32),
            pltpu.VMEM_SHARED((N, W1COLS), jnp.float32),
            pltpu.SemaphoreType.DMA,
            pltpu.SemaphoreType.DMA,
            pltpu.SemaphoreType.DMA,
            pltpu.SemaphoreType.DMA,
            pltpu.SemaphoreType.DMA,
            pltpu.SemaphoreType.DMA,
        ],
    )
    return f(src2d, dst2d, xpe1, adst_t)


def _combine1_body(acc_ref, xpe_ref, adstt_ref, b1_ref, w2_ref,
                   asrc2_ref, adst2_ref, xpe2_ref, a2s_ref, a2d_ref):
    xp = xpe_ref[:, 0:HC]
    asrc = xpe_ref[:, HC + H:W1COLS]
    adst = adstt_ref[:, 0:H]
    s = asrc + adst
    wself = jnp.exp(jnp.where(s > 0, s, 0.2 * s))
    rowh = lax.broadcasted_iota(jnp.int32, (H, HC), 0)
    colh = lax.broadcasted_iota(jnp.int32, (H, HC), 1) // C1
    R = (rowh == colh).astype(jnp.float32)
    wrep = jnp.dot(wself, R, preferred_element_type=jnp.float32)
    num = acc_ref[0, :, 0:HC] + acc_ref[1, :, 0:HC] + xp * wrep
    den = acc_ref[0, :, HC:HC + H] + acc_ref[1, :, HC:HC + H] + wself
    denr = jnp.dot(den, R, preferred_element_type=jnp.float32)
    h = num / (denr + 1e-16) + b1_ref[...][None, :]

    x2p = jnp.dot(h, w2_ref[...], preferred_element_type=jnp.float32)
    a2s = jnp.dot(x2p, asrc2_ref[...].T, preferred_element_type=jnp.float32)
    a2d = jnp.dot(x2p, adst2_ref[...].T, preferred_element_type=jnp.float32)
    xpe2_ref[:, 0:NCLASS] = x2p
    xpe2_ref[:, NCLASS:8] = jnp.zeros((N, 1), jnp.float32)
    xpe2_ref[:, 8:9] = jnp.ones((N, 1), jnp.float32)
    xpe2_ref[:, 9:16] = jnp.zeros((N, 7), jnp.float32)
    a2s_ref[...] = a2s
    a2d_ref[...] = a2d


def _combine1(acc1, xpe1, adst_t, b1, W2, a_src2, a_dst2):
    return pl.pallas_call(
        _combine1_body,
        out_shape=(
            jax.ShapeDtypeStruct((N, W2COLS), jnp.float32),
            jax.ShapeDtypeStruct((N, 1), jnp.float32),
            jax.ShapeDtypeStruct((N, 1), jnp.float32),
        ),
    )(acc1, xpe1, adst_t, b1, W2, a_src2, a_dst2)


def _edges2_body(src_hbm, dst_hbm, xpe2_hbm, a2s_hbm, a2d_hbm, acc_out,
                 src_all, dst_all, rows0, rows1, rows2, msg0, msg1, msg2,
                 a2s_v, a2d_v, zb_v, acc_s,
                 gsem0, gsem1, gsem2, ssem0, ssem1, ssem2):
    cid = lax.axis_index("c")
    sid = lax.axis_index("s")
    wid = sid * 2 + cid

    rows = (rows0, rows1, rows2)
    msg = (msg0, msg1, msg2)
    gsem = (gsem0, gsem1, gsem2)
    ssem = (ssem0, ssem1, ssem2)

    pltpu.sync_copy(a2s_hbm, a2s_v)
    pltpu.sync_copy(a2d_hbm, a2d_v)
    pltpu.sync_copy(src_hbm.at[pl.ds(wid * NCHUNK, NCHUNK)], src_all)
    pltpu.sync_copy(dst_hbm.at[pl.ds(wid * NCHUNK, NCHUNK)], dst_all)

    def _zb_row(r, _):
        zb_v[r, pl.ds(0, 16)] = jnp.zeros((16,), jnp.float32)
        return 0
    lax.fori_loop(0, ZROWS, _zb_row, 0)
    for j in range(RPT // ZROWS):
        pltpu.sync_copy(zb_v, acc_s.at[pl.ds(sid * RPT + j * ZROWS, ZROWS)])

    @pl.when(sid == 0)
    def _():
        pltpu.sync_copy(zb_v.at[pl.ds(0, 16)], acc_s.at[pl.ds(16 * RPT, 16)])
    plsc.subcore_barrier()

    def start_gather(c, b):
        pltpu.async_copy(xpe2_hbm.at[src_all.at[c]], rows[b], gsem[b])

    def chunk_step(c, b, wait_scatter, prefetch_c):
        pltpu.make_async_copy(xpe2_hbm.at[src_all.at[c]], rows[b], gsem[b]).wait()
        if wait_scatter:
            pltpu.make_async_copy(msg[b], acc_s.at[dst_all.at[c]], ssem[b]).wait()
        rows_b, msg_b = rows[b], msg[b]

        @plsc.parallel_loop(0, T // 16, unroll=1)
        def _grp(j):
            sidx = src_all[c, pl.ds(j * 16, 16)]
            didx = dst_all[c, pl.ds(j * 16, 16)]
            s = plsc.load_gather(a2s_v, [sidx]) + plsc.load_gather(a2d_v, [didx])
            s = jnp.where(s > 0, s, 0.2 * s)
            w16 = jnp.exp(s)
            e0 = j * 16
            for l in range(16):
                msg_b[e0 + l, pl.ds(0, 16)] = (
                    rows_b[e0 + l, pl.ds(0, 16)]
                    * _vperm(w16, jnp.full((16,), l, jnp.int32)))

        pltpu.async_copy(msg_b, acc_s.at[dst_all.at[c]], ssem[b], add=True)
        if prefetch_c is not None:
            start_gather(prefetch_c, b)

    start_gather(0, 0)
    start_gather(1, 1)
    start_gather(2, 2)
    chunk_step(0, 0, False, 3)
    chunk_step(1, 1, False, 4)
    chunk_step(2, 2, False, 5)

    def _trip(g, _):
        chunk_step(3 * g, 0, True, 3 * g + 3)
        chunk_step(3 * g + 1, 1, True, 3 * g + 4)
        chunk_step(3 * g + 2, 2, True, 3 * g + 5)
        return 0
    lax.fori_loop(1, 40, _trip, 0)
    chunk_step(120, 0, True, 123)
    chunk_step(121, 1, True, 124)
    chunk_step(122, 2, True, None)
    chunk_step(123, 0, True, None)
    chunk_step(124, 1, True, None)
    pltpu.make_async_copy(msg[2], acc_s.at[dst_all.at[122]], ssem[2]).wait()
    pltpu.make_async_copy(msg[0], acc_s.at[dst_all.at[123]], ssem[0]).wait()
    pltpu.make_async_copy(msg[1], acc_s.at[dst_all.at[124]], ssem[1]).wait()

    plsc.subcore_barrier()
    pltpu.sync_copy(acc_s.at[pl.ds(sid * RPT, RPT)],
                    acc_out.at[cid, pl.ds(sid * RPT, RPT)])

    @pl.when(sid == 0)
    def _():
        pltpu.sync_copy(acc_s.at[pl.ds(16 * RPT, 16)],
                        acc_out.at[cid, pl.ds(16 * RPT, 16)])


def _edges2(src2d, dst2d, xpe2, a2s, a2d):
    mesh = plsc.VectorSubcoreMesh(core_axis_name="c", subcore_axis_name="s")
    f = pl.kernel(
        _edges2_body,
        out_type=jax.ShapeDtypeStruct((2, N, W2COLS), jnp.float32),
        mesh=mesh,
        compiler_params=pltpu.CompilerParams(
            use_tc_tiling_on_sc=False, needs_layout_passes=False),
        scratch_types=[
            pltpu.VMEM((NCHUNK, T), jnp.int32),
            pltpu.VMEM((NCHUNK, T), jnp.int32),
            pltpu.VMEM((T, W2COLS), jnp.float32),
            pltpu.VMEM((T, W2COLS), jnp.float32),
            pltpu.VMEM((T, W2COLS), jnp.float32),
            pltpu.VMEM((T, W2COLS), jnp.float32),
            pltpu.VMEM((T, W2COLS), jnp.float32),
            pltpu.VMEM((T, W2COLS), jnp.float32),
            pltpu.VMEM((N,), jnp.float32),
            pltpu.VMEM((N,), jnp.float32),
            pltpu.VMEM((ZROWS, W2COLS), jnp.float32),
            pltpu.VMEM_SHARED((N, W2COLS), jnp.float32),
            pltpu.SemaphoreType.DMA,
            pltpu.SemaphoreType.DMA,
            pltpu.SemaphoreType.DMA,
            pltpu.SemaphoreType.DMA,
            pltpu.SemaphoreType.DMA,
            pltpu.SemaphoreType.DMA,
        ],
    )
    return f(src2d, dst2d, xpe2, a2s, a2d)


def _final_body(acc_ref, xpe2_ref, a2s_ref, a2d_ref, b2_ref, out_ref):
    x2p = xpe2_ref[:, 0:NCLASS]
    s = a2s_ref[...] + a2d_ref[...]
    wself = jnp.exp(jnp.where(s > 0, s, 0.2 * s))
    num = acc_ref[0, :, 0:NCLASS] + acc_ref[1, :, 0:NCLASS] + x2p * wself
    den = acc_ref[0, :, 8:9] + acc_ref[1, :, 8:9] + wself
    o = num / (den + 1e-16) + b2_ref[...][None, :]
    o = jnp.where(o > 0, o, jnp.exp(o) - 1.0)
    m = jnp.max(o, axis=1, keepdims=True)
    t = o - m
    lse = jnp.log(jnp.sum(jnp.exp(t), axis=1, keepdims=True))
    out_ref[...] = t - lse


def _final(acc2, xpe2, a2s, a2d, b2):
    return pl.pallas_call(
        _final_body,
        out_shape=jax.ShapeDtypeStruct((N, NCLASS), jnp.float32),
    )(acc2, xpe2, a2s, a2d, b2)


def kernel(x, edge_index, W1, a_src1, a_dst1, b1, W2, a_src2, a_dst2, b2):
    src2d = edge_index[0].reshape(NWORKERS * NCHUNK, T)
    dst2d = edge_index[1].reshape(NWORKERS * NCHUNK, T)
    xpe1, adst_t = _prep1(x, W1, a_src1, a_dst1)
    acc1 = _edges1(src2d, dst2d, xpe1, adst_t)
    xpe2, a2s, a2d = _combine1(acc1, xpe1, adst_t, b1, W2, a_src2, a_dst2)
    acc2 = _edges2(src2d, dst2d, xpe2, a2s.reshape(N), a2d.reshape(N))
    return _final(acc2, xpe2, a2s, a2d, b2)
